# baseline (speedup 1.0000x reference)
import functools

import jax
import jax.numpy as jnp
from jax import lax
from jax.experimental import pallas as pl
from jax.experimental.pallas import tpu as pltpu
from jax.experimental.pallas import tpu_sc as plsc

L = 16
NW = 32

_DNUMS = lax.GatherDimensionNumbers(
    offset_dims=(), collapsed_slice_dims=(0,), start_index_map=(0,)
)


def _perm(v, idx):
    return lax.gather(
        v, idx[:, None], _DNUMS, (1,),
        mode=lax.GatherScatterMode.PROMISE_IN_BOUNDS,
    )


def _consts16():
    lane = lax.iota(jnp.int32, L)
    kmask = {k: (lane & k) == 0 for k in (1, 2, 4, 8)}
    xidx = {k: lane ^ k for k in (1, 2, 4, 8)}
    return kmask, xidx


def _transpose16(vs, kmask, xidx):
    for k in (1, 2, 4, 8):
        nvs = list(vs)
        km = kmask[k]
        for a0 in range(L):
            if a0 & k:
                continue
            p = a0 | k
            a, b = vs[a0], vs[p]
            pc = _perm(jnp.where(km, b, a), xidx[k])
            nvs[a0] = jnp.where(km, a, pc)
            nvs[p] = jnp.where(km, pc, b)
        vs = nvs
    return vs


def _make_table_linearize(V4, D):
    RG = V4 // 128
    eg_n = D // 8
    mesh = plsc.VectorSubcoreMesh(core_axis_name="c", subcore_axis_name="s")

    @functools.partial(
        pl.kernel,
        mesh=mesh,
        compiler_params=pltpu.CompilerParams(use_tc_tiling_on_sc=False),
        out_type=[jax.ShapeDtypeStruct((V4, D), jnp.float32)],
        scratch_types=[
            pltpu.VMEM((D, 128), jnp.float32),
            pltpu.VMEM((D, 128), jnp.float32),
            pltpu.VMEM((128, D), jnp.float32),
            pltpu.VMEM((128, D), jnp.float32),
            pltpu.SemaphoreType.DMA,
            pltpu.SemaphoreType.DMA,
            pltpu.SemaphoreType.DMA,
            pltpu.SemaphoreType.DMA,
        ],
    )
    def lin(tab4_hbm, out_hbm, t0, t1, o0, o1, si0, si1, so0, so1):
        tile_v = [t0, t1]
        lin_v = [o0, o1]
        sem_i = [si0, si1]
        sem_o = [so0, so1]
        wid = lax.axis_index("s") * 2 + lax.axis_index("c")
        n_mine = (RG - wid + NW - 1) // NW
        kmask, xidx = _consts16()

        def start(j, s):
            rg = wid + j * NW

            @pl.when(rg < RG)
            def _():
                for eg in range(eg_n):
                    pltpu.async_copy(
                        tab4_hbm.at[eg, rg],
                        tile_v[s].at[pl.ds(eg * 8, 8)],
                        sem_i[s],
                    )

        def wait_out(s):
            pltpu.make_async_copy(out_hbm.at[pl.ds(0, 128)], lin_v[s], sem_o[s]).wait()

        start(0, 0)

        def body(j, carry):
            for s in (0, 1):
                jj = j * 2 + s
                rg = wid + jj * NW

                @pl.when(jj + 1 < n_mine)
                def _pf():
                    @pl.when(jj >= 1)
                    def _():
                        wait_out(1 - s)

                    start(jj + 1, 1 - s)

                @pl.when(rg < RG)
                def _work():
                    pltpu.make_async_copy(
                        tab4_hbm.at[0, 0], tile_v[s].at[pl.ds(0, 8)], sem_i[s]
                    ).wait()
                    pltpu.make_async_copy(
                        tab4_hbm.at[0, 0], tile_v[s].at[pl.ds(8, 8)], sem_i[s]
                    ).wait()

                    for c in range(8):
                        vs = [tile_v[s][r, pl.ds(c * L, L)] for r in range(D)]
                        vs = _transpose16(vs, kmask, xidx)
                        for r in range(L):
                            lin_v[s][c * L + r] = vs[r]

                    pltpu.async_copy(
                        lin_v[s], out_hbm.at[pl.ds(rg * 128, 128)], sem_o[s]
                    )
            return carry

        lax.fori_loop(0, (n_mine + 1) // 2, body, 0)

        @pl.when(n_mine >= 2)
        def _():
            wait_out(1)

        @pl.when(n_mine >= 1)
        def _():
            wait_out(0)

    return lin


def _make_sc_embed(TU, B, D, T, TIME):
    n_iter = (TU + NW - 1) // NW
    eg_n, bg_n = D // 8, B // 128
    ug_n = (TIME + 7) // 8
    mesh = plsc.VectorSubcoreMesh(core_axis_name="c", subcore_axis_name="s")

    @functools.partial(
        pl.kernel,
        mesh=mesh,
        compiler_params=pltpu.CompilerParams(use_tc_tiling_on_sc=False),
        out_type=[
            jax.ShapeDtypeStruct((TU, eg_n, bg_n, 8, 128), jnp.float32),
            jax.ShapeDtypeStruct((T, ug_n, bg_n, 8, 128), jnp.float32),
        ],
        scratch_types=[
            pltpu.VMEM((bg_n, 128), jnp.int32),
            pltpu.VMEM((bg_n, 128), jnp.int32),
            pltpu.VMEM((B, D), jnp.float32),
            pltpu.VMEM((B, D), jnp.float32),
            pltpu.VMEM((eg_n, bg_n, 8, 128), jnp.float32),
            pltpu.VMEM((eg_n, bg_n, 8, 128), jnp.float32),
            pltpu.VMEM((bg_n, 128), jnp.float32),
            pltpu.VMEM((bg_n, 128), jnp.float32),
            pltpu.SemaphoreType.DMA,
            pltpu.SemaphoreType.DMA,
            pltpu.SemaphoreType.DMA,
            pltpu.SemaphoreType.DMA,
        ],
    )
    def sc_embed(
        idx_hbm, table_hbm, out_hbm, mask_hbm,
        idx0, idx1, rows0, rows1, pb0, pb1, msk0, msk1,
        sg0, sg1, so0, so1,
    ):
        idx_v = [idx0, idx1]
        rows_v = [rows0, rows1]
        pb_v = [pb0, pb1]
        mask_v = [msk0, msk1]
        sem_g = [sg0, sg1]
        sem_o = [so0, so1]
        wid = lax.axis_index("s") * 2 + lax.axis_index("c")
        n_mine = (TU - wid + NW - 1) // NW
        kmask, xidx = _consts16()

        def start_unit(j, s):
            tu = wid + j * NW

            @pl.when(tu < TU)
            def _():
                tt = tu // TIME
                u = tu % TIME
                pltpu.sync_copy(idx_hbm.at[tt, u // 8, :, u % 8, :], idx_v[s])
                for bg in range(bg_n):
                    pltpu.async_copy(
                        table_hbm.at[idx_v[s].at[bg]],
                        rows_v[s].at[pl.ds(bg * 128, 128)],
                        sem_g[s],
                    )

        def wait_out(s):
            pltpu.make_async_copy(out_hbm.at[0], pb_v[s], sem_o[s]).wait()
            pltpu.make_async_copy(mask_hbm.at[0, 0, :, 0, :], mask_v[s], sem_o[s]).wait()

        start_unit(0, 0)

        def unit_body(j, carry):
            for s in (0, 1):
                jj = j * 2 + s
                tu = wid + jj * NW

                @pl.when(jj + 1 < n_mine)
                def _pf():
                    @pl.when(jj >= 1)
                    def _():
                        wait_out(1 - s)

                    start_unit(jj + 1, 1 - s)

                @pl.when(tu < TU)
                def _work():
                    pltpu.make_async_copy(
                        table_hbm.at[pl.ds(0, B)], rows_v[s], sem_g[s]
                    ).wait()

                    def blk(i, c):
                        b0 = i * L
                        bgi = i >> 3
                        off = (i & 7) * L
                        iv = idx_v[s][bgi, pl.ds(off, L)]
                        m = jnp.where(iv == 0, 0.0, 1.0)
                        mask_v[s][bgi, pl.ds(off, L)] = m

                        vs = [rows_v[s][b0 + r] for r in range(L)]
                        vs = _transpose16(vs, kmask, xidx)

                        for e in range(D):
                            pb_v[s][e // 8, bgi, e % 8, pl.ds(off, L)] = vs[e] * m
                        return c

                    lax.fori_loop(0, B // L, blk, 0)

                    tt = tu // TIME
                    u = tu % TIME
                    pltpu.async_copy(pb_v[s], out_hbm.at[tu], sem_o[s])
                    pltpu.async_copy(
                        mask_v[s], mask_hbm.at[tt, u // 8, :, u % 8, :], sem_o[s]
                    )
            return carry

        lax.fori_loop(0, (n_iter + 1) // 2, unit_body, 0)

        @pl.when(n_mine >= 2)
        def _():
            wait_out(1)

        wait_out(0)

    return sc_embed


def kernel(x, table):
    b, t, time = x.shape
    v, d = table.shape
    tu = t * time
    up = (-time) % 8
    tp = time + up
    xq = jnp.pad(x.astype(jnp.int32), ((0, 0), (0, 0), (0, up)))
    xk = (
        jnp.transpose(xq, (1, 2, 0))
        .reshape(t, tp // 8, 8, b // 128, 128)
        .transpose(0, 1, 3, 2, 4)
    )
    vp = (-v) % 128
    v4 = v + vp
    tq = jnp.pad(table, ((0, vp), (0, 0)))
    tab4 = (
        jnp.transpose(tq, (1, 0))
        .reshape(d // 8, 8, v4 // 128, 128)
        .transpose(0, 2, 1, 3)
    )
    (tab_lin,) = _make_table_linearize(v4, d)(tab4)
    A, maskM = _make_sc_embed(tu, b, d, t, time)(xk, tab_lin)
    emb = (
        A.reshape(t, time, d // 8, b // 128, 8, 128)
        .transpose(3, 5, 0, 1, 2, 4)
        .reshape(b, t, time, d)
    )
    mask = (
        maskM.transpose(0, 1, 3, 2, 4)
        .reshape(t, tp, b)
        .transpose(2, 0, 1)[:, :, :time]
    )
    return emb, mask


# speedup vs baseline: 1.0208x; 1.0208x over previous
import functools

import jax
import jax.numpy as jnp
from jax import lax
from jax.experimental import pallas as pl
from jax.experimental.pallas import tpu as pltpu
from jax.experimental.pallas import tpu_sc as plsc

L = 16
NW = 32


def _make_table_linearize(V4, D):
    RG = V4 // 128
    eg_n = D // 8
    mesh = plsc.VectorSubcoreMesh(core_axis_name="c", subcore_axis_name="s")

    @functools.partial(
        pl.kernel,
        mesh=mesh,
        compiler_params=pltpu.CompilerParams(use_tc_tiling_on_sc=False),
        out_type=[jax.ShapeDtypeStruct((V4, D), jnp.float32)],
        scratch_types=[
            pltpu.VMEM((D, 128), jnp.float32),
            pltpu.VMEM((D, 128), jnp.float32),
            pltpu.VMEM((128, D), jnp.float32),
            pltpu.VMEM((128, D), jnp.float32),
            pltpu.SemaphoreType.DMA,
            pltpu.SemaphoreType.DMA,
            pltpu.SemaphoreType.DMA,
            pltpu.SemaphoreType.DMA,
        ],
    )
    def lin(tab4_hbm, out_hbm, t0, t1, o0, o1, si0, si1, so0, so1):
        tile_v = [t0, t1]
        lin_v = [o0, o1]
        sem_i = [si0, si1]
        sem_o = [so0, so1]
        wid = lax.axis_index("s") * 2 + lax.axis_index("c")
        n_mine = (RG - wid + NW - 1) // NW

        lane = lax.iota(jnp.int32, L)
        kmask = {k: (lane & k) == 0 for k in (1, 2, 4, 8)}
        rot_r_idx = {k: (lane - k) & (L - 1) for k in (1, 2, 4, 8)}
        rot_l_idx = {k: (lane + k) & (L - 1) for k in (1, 2, 4, 8)}

        dnums = lax.GatherDimensionNumbers(
            offset_dims=(), collapsed_slice_dims=(0,), start_index_map=(0,)
        )

        def _perm(v, idx):
            return lax.gather(
                v, idx[:, None], dnums, (1,),
                mode=lax.GatherScatterMode.PROMISE_IN_BOUNDS,
            )

        def start(j, s):
            rg = wid + j * NW

            @pl.when(rg < RG)
            def _():
                for eg in range(eg_n):
                    pltpu.async_copy(
                        tab4_hbm.at[eg, rg],
                        tile_v[s].at[pl.ds(eg * 8, 8)],
                        sem_i[s],
                    )

        def wait_out(s):
            pltpu.make_async_copy(out_hbm.at[pl.ds(0, 128)], lin_v[s], sem_o[s]).wait()

        start(0, 0)

        def body(j, carry):
            for s in (0, 1):
                jj = j * 2 + s
                rg = wid + jj * NW

                @pl.when(jj + 1 < n_mine)
                def _pf():
                    @pl.when(jj >= 1)
                    def _():
                        wait_out(1 - s)

                    start(jj + 1, 1 - s)

                @pl.when(rg < RG)
                def _work():
                    pltpu.make_async_copy(
                        tab4_hbm.at[0, 0], tile_v[s].at[pl.ds(0, 8)], sem_i[s]
                    ).wait()
                    pltpu.make_async_copy(
                        tab4_hbm.at[0, 0], tile_v[s].at[pl.ds(8, 8)], sem_i[s]
                    ).wait()

                    for c in range(8):
                        vs = [tile_v[s][r, pl.ds(c * L, L)] for r in range(D)]
                        for k in (1, 2, 4, 8):
                            nvs = list(vs)
                            km = kmask[k]
                            for a0 in range(L):
                                if a0 & k:
                                    continue
                                p = a0 | k
                                a, b = vs[a0], vs[p]
                                nvs[a0] = jnp.where(km, a, _perm(b, rot_r_idx[k]))
                                nvs[p] = jnp.where(km, _perm(a, rot_l_idx[k]), b)
                            vs = nvs
                        for r in range(L):
                            lin_v[s][c * L + r] = vs[r]

                    pltpu.async_copy(
                        lin_v[s], out_hbm.at[pl.ds(rg * 128, 128)], sem_o[s]
                    )
            return carry

        lax.fori_loop(0, (n_mine + 1) // 2, body, 0)

        @pl.when(n_mine >= 2)
        def _():
            wait_out(1)

        @pl.when(n_mine >= 1)
        def _():
            wait_out(0)

    return lin


def _make_sc_embed(TU, B, D, T, TIME):
    n_iter = (TU + NW - 1) // NW
    eg_n, bg_n = D // 8, B // 128
    ug_n = (TIME + 7) // 8
    mesh = plsc.VectorSubcoreMesh(core_axis_name="c", subcore_axis_name="s")

    @functools.partial(
        pl.kernel,
        mesh=mesh,
        compiler_params=pltpu.CompilerParams(use_tc_tiling_on_sc=False),
        out_type=[
            jax.ShapeDtypeStruct((TU, eg_n, bg_n, 8, 128), jnp.float32),
            jax.ShapeDtypeStruct((T, ug_n, bg_n, 8, 128), jnp.float32),
        ],
        scratch_types=[
            pltpu.VMEM((bg_n, 128), jnp.int32),
            pltpu.VMEM((bg_n, 128), jnp.int32),
            pltpu.VMEM((B, D), jnp.float32),
            pltpu.VMEM((B, D), jnp.float32),
            pltpu.VMEM((eg_n, bg_n, 8, 128), jnp.float32),
            pltpu.VMEM((eg_n, bg_n, 8, 128), jnp.float32),
            pltpu.VMEM((bg_n, 128), jnp.float32),
            pltpu.VMEM((bg_n, 128), jnp.float32),
            pltpu.SemaphoreType.DMA,
            pltpu.SemaphoreType.DMA,
            pltpu.SemaphoreType.DMA,
            pltpu.SemaphoreType.DMA,
        ],
    )
    def sc_embed(
        idx_hbm, table_hbm, out_hbm, mask_hbm,
        idx0, idx1, rows0, rows1, pb0, pb1, msk0, msk1,
        sg0, sg1, so0, so1,
    ):
        idx_v = [idx0, idx1]
        rows_v = [rows0, rows1]
        pb_v = [pb0, pb1]
        mask_v = [msk0, msk1]
        sem_g = [sg0, sg1]
        sem_o = [so0, so1]
        wid = lax.axis_index("s") * 2 + lax.axis_index("c")
        n_mine = (TU - wid + NW - 1) // NW

        lane = lax.iota(jnp.int32, L)
        kmask = {k: (lane & k) == 0 for k in (1, 2, 4, 8)}
        rot_r_idx = {k: (lane - k) & (L - 1) for k in (1, 2, 4, 8)}
        rot_l_idx = {k: (lane + k) & (L - 1) for k in (1, 2, 4, 8)}

        def _perm(v, idx):
            dnums = lax.GatherDimensionNumbers(
                offset_dims=(), collapsed_slice_dims=(0,), start_index_map=(0,)
            )
            return lax.gather(
                v, idx[:, None], dnums, (1,),
                mode=lax.GatherScatterMode.PROMISE_IN_BOUNDS,
            )

        def start_unit(j, s):
            tu = wid + j * NW

            @pl.when(tu < TU)
            def _():
                tt = tu // TIME
                u = tu % TIME
                pltpu.sync_copy(idx_hbm.at[tt, u // 8, :, u % 8, :], idx_v[s])
                for bg in range(bg_n):
                    pltpu.async_copy(
                        table_hbm.at[idx_v[s].at[bg]],
                        rows_v[s].at[pl.ds(bg * 128, 128)],
                        sem_g[s],
                    )

        def wait_out(s):
            pltpu.make_async_copy(out_hbm.at[0], pb_v[s], sem_o[s]).wait()
            pltpu.make_async_copy(mask_hbm.at[0, 0, :, 0, :], mask_v[s], sem_o[s]).wait()

        start_unit(0, 0)

        def unit_body(j, carry):
            for s in (0, 1):
                jj = j * 2 + s
                tu = wid + jj * NW

                @pl.when(jj + 1 < n_mine)
                def _pf():
                    @pl.when(jj >= 1)
                    def _():
                        wait_out(1 - s)

                    start_unit(jj + 1, 1 - s)

                @pl.when(tu < TU)
                def _work():
                    pltpu.make_async_copy(
                        table_hbm.at[pl.ds(0, B)], rows_v[s], sem_g[s]
                    ).wait()

                    def one_blk(i):
                        b0 = i * L
                        bgi = i >> 3
                        off = (i & 7) * L
                        iv = idx_v[s][bgi, pl.ds(off, L)]
                        m = jnp.where(iv == 0, 0.0, 1.0)
                        mask_v[s][bgi, pl.ds(off, L)] = m

                        vs = [rows_v[s][b0 + r] for r in range(L)]
                        for k in (1, 2, 4, 8):
                            nvs = list(vs)
                            km = kmask[k]
                            for a0 in range(L):
                                if a0 & k:
                                    continue
                                p = a0 | k
                                a, b = vs[a0], vs[p]
                                nvs[a0] = jnp.where(km, a, _perm(b, rot_r_idx[k]))
                                nvs[p] = jnp.where(km, _perm(a, rot_l_idx[k]), b)
                            vs = nvs

                        for e in range(D):
                            pb_v[s][e // 8, bgi, e % 8, pl.ds(off, L)] = vs[e] * m

                    def blk(i2, c):
                        one_blk(i2 * 2)
                        one_blk(i2 * 2 + 1)
                        return c

                    lax.fori_loop(0, B // L // 2, blk, 0)

                    tt = tu // TIME
                    u = tu % TIME
                    pltpu.async_copy(pb_v[s], out_hbm.at[tu], sem_o[s])
                    pltpu.async_copy(
                        mask_v[s], mask_hbm.at[tt, u // 8, :, u % 8, :], sem_o[s]
                    )
            return carry

        lax.fori_loop(0, (n_iter + 1) // 2, unit_body, 0)

        @pl.when(n_mine >= 2)
        def _():
            wait_out(1)

        wait_out(0)

    return sc_embed


def kernel(x, table):
    b, t, time = x.shape
    v, d = table.shape
    tu = t * time
    up = (-time) % 8
    tp = time + up
    xq = jnp.pad(x.astype(jnp.int32), ((0, 0), (0, 0), (0, up)))
    xk = (
        jnp.transpose(xq, (1, 2, 0))
        .reshape(t, tp // 8, 8, b // 128, 128)
        .transpose(0, 1, 3, 2, 4)
    )
    vp = (-v) % 128
    v4 = v + vp
    tq = jnp.pad(table, ((0, vp), (0, 0)))
    tab4 = (
        jnp.transpose(tq, (1, 0))
        .reshape(d // 8, 8, v4 // 128, 128)
        .transpose(0, 2, 1, 3)
    )
    (tab_lin,) = _make_table_linearize(v4, d)(tab4)
    A, maskM = _make_sc_embed(tu, b, d, t, time)(xk, tab_lin)
    emb = (
        A.reshape(t, time, d // 8, b // 128, 8, 128)
        .transpose(3, 5, 0, 1, 2, 4)
        .reshape(b, t, time, d)
    )
    mask = (
        maskM.transpose(0, 1, 3, 2, 4)
        .reshape(t, tp, b)
        .transpose(2, 0, 1)[:, :, :time]
    )
    return emb, mask


# speedup vs baseline: 1.0211x; 1.0002x over previous
import functools

import jax
import jax.numpy as jnp
from jax import lax
from jax.experimental import pallas as pl
from jax.experimental.pallas import tpu as pltpu
from jax.experimental.pallas import tpu_sc as plsc

L = 16
NW = 32


def _make_table_linearize(V4, D):
    RG = V4 // 128
    eg_n = D // 8
    mesh = plsc.VectorSubcoreMesh(core_axis_name="c", subcore_axis_name="s")

    @functools.partial(
        pl.kernel,
        mesh=mesh,
        compiler_params=pltpu.CompilerParams(use_tc_tiling_on_sc=False),
        out_type=[jax.ShapeDtypeStruct((V4, D), jnp.float32)],
        scratch_types=[
            pltpu.VMEM((D, 128), jnp.float32),
            pltpu.VMEM((D, 128), jnp.float32),
            pltpu.VMEM((128, D), jnp.float32),
            pltpu.VMEM((128, D), jnp.float32),
            pltpu.SemaphoreType.DMA,
            pltpu.SemaphoreType.DMA,
            pltpu.SemaphoreType.DMA,
            pltpu.SemaphoreType.DMA,
        ],
    )
    def lin(tab4_hbm, out_hbm, t0, t1, o0, o1, si0, si1, so0, so1):
        tile_v = [t0, t1]
        lin_v = [o0, o1]
        sem_i = [si0, si1]
        sem_o = [so0, so1]
        wid = lax.axis_index("s") * 2 + lax.axis_index("c")
        n_mine = (RG - wid + NW - 1) // NW

        lane = lax.iota(jnp.int32, L)
        kmask = {k: (lane & k) == 0 for k in (1, 2, 4, 8)}
        rot_r_idx = {k: (lane - k) & (L - 1) for k in (1, 2, 4, 8)}
        rot_l_idx = {k: (lane + k) & (L - 1) for k in (1, 2, 4, 8)}

        dnums = lax.GatherDimensionNumbers(
            offset_dims=(), collapsed_slice_dims=(0,), start_index_map=(0,)
        )

        def _perm(v, idx):
            return lax.gather(
                v, idx[:, None], dnums, (1,),
                mode=lax.GatherScatterMode.PROMISE_IN_BOUNDS,
            )

        def start(j, s):
            rg = wid + j * NW

            @pl.when(rg < RG)
            def _():
                for eg in range(eg_n):
                    pltpu.async_copy(
                        tab4_hbm.at[eg, rg],
                        tile_v[s].at[pl.ds(eg * 8, 8)],
                        sem_i[s],
                    )

        def wait_out(s):
            pltpu.make_async_copy(out_hbm.at[pl.ds(0, 128)], lin_v[s], sem_o[s]).wait()

        start(0, 0)

        def body(j, carry):
            for s in (0, 1):
                jj = j * 2 + s
                rg = wid + jj * NW

                @pl.when(jj + 1 < n_mine)
                def _pf():
                    @pl.when(jj >= 1)
                    def _():
                        wait_out(1 - s)

                    start(jj + 1, 1 - s)

                @pl.when(rg < RG)
                def _work():
                    pltpu.make_async_copy(
                        tab4_hbm.at[0, 0], tile_v[s].at[pl.ds(0, 8)], sem_i[s]
                    ).wait()
                    pltpu.make_async_copy(
                        tab4_hbm.at[0, 0], tile_v[s].at[pl.ds(8, 8)], sem_i[s]
                    ).wait()

                    for c in range(8):
                        vs = [tile_v[s][r, pl.ds(c * L, L)] for r in range(D)]
                        for k in (1, 2, 4, 8):
                            nvs = list(vs)
                            km = kmask[k]
                            for a0 in range(L):
                                if a0 & k:
                                    continue
                                p = a0 | k
                                a, b = vs[a0], vs[p]
                                nvs[a0] = jnp.where(km, a, _perm(b, rot_r_idx[k]))
                                nvs[p] = jnp.where(km, _perm(a, rot_l_idx[k]), b)
                            vs = nvs
                        for r in range(L):
                            lin_v[s][c * L + r] = vs[r]

                    pltpu.async_copy(
                        lin_v[s], out_hbm.at[pl.ds(rg * 128, 128)], sem_o[s]
                    )
            return carry

        lax.fori_loop(0, (n_mine + 1) // 2, body, 0)

        @pl.when(n_mine >= 2)
        def _():
            wait_out(1)

        @pl.when(n_mine >= 1)
        def _():
            wait_out(0)

    return lin


def _make_sc_embed(TU, B, D, T, TIME):
    n_iter = (TU + NW - 1) // NW
    eg_n, bg_n = D // 8, B // 128
    ug_n = (TIME + 7) // 8
    mesh = plsc.VectorSubcoreMesh(core_axis_name="c", subcore_axis_name="s")

    @functools.partial(
        pl.kernel,
        mesh=mesh,
        compiler_params=pltpu.CompilerParams(use_tc_tiling_on_sc=False),
        out_type=[
            jax.ShapeDtypeStruct((TU, eg_n, bg_n, 8, 128), jnp.float32),
            jax.ShapeDtypeStruct((T, ug_n, bg_n, 8, 128), jnp.float32),
        ],
        scratch_types=[
            pltpu.VMEM((bg_n, 128), jnp.int32),
            pltpu.VMEM((bg_n, 128), jnp.int32),
            pltpu.VMEM((B, D), jnp.float32),
            pltpu.VMEM((B, D), jnp.float32),
            pltpu.VMEM((eg_n, bg_n, 8, 128), jnp.float32),
            pltpu.VMEM((eg_n, bg_n, 8, 128), jnp.float32),
            pltpu.VMEM((bg_n, 128), jnp.float32),
            pltpu.VMEM((bg_n, 128), jnp.float32),
            pltpu.SemaphoreType.DMA,
            pltpu.SemaphoreType.DMA,
            pltpu.SemaphoreType.DMA,
            pltpu.SemaphoreType.DMA,
        ],
    )
    def sc_embed(
        idx_hbm, table_hbm, out_hbm, mask_hbm,
        idx0, idx1, rows0, rows1, pb0, pb1, msk0, msk1,
        sg0, sg1, so0, so1,
    ):
        idx_v = [idx0, idx1]
        rows_v = [rows0, rows1]
        pb_v = [pb0, pb1]
        mask_v = [msk0, msk1]
        sem_g = [sg0, sg1]
        sem_o = [so0, so1]
        wid = lax.axis_index("s") * 2 + lax.axis_index("c")
        n_mine = (TU - wid + NW - 1) // NW

        lane = lax.iota(jnp.int32, L)
        kmask = {k: (lane & k) == 0 for k in (1, 2, 4, 8)}
        rot_r_idx = {k: (lane - k) & (L - 1) for k in (1, 2, 4, 8)}
        rot_l_idx = {k: (lane + k) & (L - 1) for k in (1, 2, 4, 8)}

        def _perm(v, idx):
            dnums = lax.GatherDimensionNumbers(
                offset_dims=(), collapsed_slice_dims=(0,), start_index_map=(0,)
            )
            return lax.gather(
                v, idx[:, None], dnums, (1,),
                mode=lax.GatherScatterMode.PROMISE_IN_BOUNDS,
            )

        def start_unit(j, s):
            tu = wid + j * NW

            @pl.when(tu < TU)
            def _():
                tt = tu // TIME
                u = tu % TIME
                pltpu.sync_copy(idx_hbm.at[tt, u // 8, :, u % 8, :], idx_v[s])
                for bg in range(bg_n):
                    pltpu.async_copy(
                        table_hbm.at[idx_v[s].at[bg]],
                        rows_v[s].at[pl.ds(bg * 128, 128)],
                        sem_g[s],
                    )

        def wait_out(s):
            pltpu.make_async_copy(out_hbm.at[0], pb_v[s], sem_o[s]).wait()
            pltpu.make_async_copy(mask_hbm.at[0, 0, :, 0, :], mask_v[s], sem_o[s]).wait()

        start_unit(0, 0)

        def unit_body(j, carry):
            for s in (0, 1):
                jj = j * 2 + s
                tu = wid + jj * NW

                @pl.when(jj + 1 < n_mine)
                def _pf():
                    @pl.when(jj >= 1)
                    def _():
                        wait_out(1 - s)

                    start_unit(jj + 1, 1 - s)

                @pl.when(tu < TU)
                def _work():
                    pltpu.make_async_copy(
                        table_hbm.at[pl.ds(0, B)], rows_v[s], sem_g[s]
                    ).wait()

                    def one_blk(i):
                        b0 = i * L
                        bgi = i >> 3
                        off = (i & 7) * L
                        iv = idx_v[s][bgi, pl.ds(off, L)]
                        m = jnp.where(iv == 0, 0.0, 1.0)
                        mask_v[s][bgi, pl.ds(off, L)] = m

                        vs = [rows_v[s][b0 + r] for r in range(L)]
                        for k in (1, 2, 4, 8):
                            nvs = list(vs)
                            km = kmask[k]
                            for a0 in range(L):
                                if a0 & k:
                                    continue
                                p = a0 | k
                                a, b = vs[a0], vs[p]
                                nvs[a0] = jnp.where(km, a, _perm(b, rot_r_idx[k]))
                                nvs[p] = jnp.where(km, _perm(a, rot_l_idx[k]), b)
                            vs = nvs

                        for e in range(D):
                            pb_v[s][e // 8, bgi, e % 8, pl.ds(off, L)] = vs[e] * m

                    def blk(i2, c):
                        for q in range(4):
                            one_blk(i2 * 4 + q)
                        return c

                    lax.fori_loop(0, B // L // 4, blk, 0)

                    tt = tu // TIME
                    u = tu % TIME
                    pltpu.async_copy(pb_v[s], out_hbm.at[tu], sem_o[s])
                    pltpu.async_copy(
                        mask_v[s], mask_hbm.at[tt, u // 8, :, u % 8, :], sem_o[s]
                    )
            return carry

        lax.fori_loop(0, (n_iter + 1) // 2, unit_body, 0)

        @pl.when(n_mine >= 2)
        def _():
            wait_out(1)

        wait_out(0)

    return sc_embed


def kernel(x, table):
    b, t, time = x.shape
    v, d = table.shape
    tu = t * time
    up = (-time) % 8
    tp = time + up
    xq = jnp.pad(x.astype(jnp.int32), ((0, 0), (0, 0), (0, up)))
    xk = (
        jnp.transpose(xq, (1, 2, 0))
        .reshape(t, tp // 8, 8, b // 128, 128)
        .transpose(0, 1, 3, 2, 4)
    )
    vp = (-v) % 128
    v4 = v + vp
    tq = jnp.pad(table, ((0, vp), (0, 0)))
    tab4 = (
        jnp.transpose(tq, (1, 0))
        .reshape(d // 8, 8, v4 // 128, 128)
        .transpose(0, 2, 1, 3)
    )
    (tab_lin,) = _make_table_linearize(v4, d)(tab4)
    A, maskM = _make_sc_embed(tu, b, d, t, time)(xk, tab_lin)
    emb = (
        A.reshape(t, time, d // 8, b // 128, 8, 128)
        .transpose(3, 5, 0, 1, 2, 4)
        .reshape(b, t, time, d)
    )
    mask = (
        maskM.transpose(0, 1, 3, 2, 4)
        .reshape(t, tp, b)
        .transpose(2, 0, 1)[:, :, :time]
    )
    return emb, mask
